# baseline (speedup 1.0000x reference)
import functools

import jax
import jax.numpy as jnp
from jax import lax
from jax.experimental import pallas as pl
from jax.experimental.pallas import tpu as pltpu
from jax.experimental.pallas import tpu_sc as plsc

B = 1024
S = 200
D = 64
N = B * S
V = 1000000
NC = 2
NS = 16
NW = NC * NS
W = 128
NB = V // W
TAIL = V - NB * W
BPW = (NB + NW - 1) // NW
CHUNK = 128
PER_W = N // NW
CHUNKS_W = PER_W // CHUNK

_mesh = plsc.VectorSubcoreMesh(core_axis_name="c", subcore_axis_name="s")
_params = pltpu.CompilerParams(
    use_tc_tiling_on_sc=True, needs_layout_passes=False)


@functools.partial(
    pl.kernel,
    mesh=_mesh,
    compiler_params=_params,
    out_type=jax.ShapeDtypeStruct((V, 2 * D), jnp.float32),
    scratch_types=[
        pltpu.VMEM((2, D, W), jnp.float32),
        pltpu.VMEM((2, W, 2 * D), jnp.float32),
        pltpu.VMEM((D, TAIL), jnp.float32),
        pltpu.SemaphoreType.DMA,
        pltpu.SemaphoreType.DMA,
        pltpu.SemaphoreType.DMA,
        pltpu.SemaphoreType.DMA,
    ],
)
def _format(wt_hbm, fmt_hbm, ibufs, obufs, tbuf, gi0, gi1, go0, go1):
    wid = lax.axis_index("s") * NC + lax.axis_index("c")
    c0 = wid * BPW
    gisem = (gi0, gi1)
    gosem = (go0, go1)
    lanes = lax.iota(jnp.int32, 16)

    def in_copy(c, b):
        return pltpu.make_async_copy(
            wt_hbm.at[:, pl.ds(c * W, W)], ibufs.at[b], gisem[b])

    def out_copy(c, b):
        return pltpu.make_async_copy(
            obufs.at[b], fmt_hbm.at[pl.ds(c * W, W)], gosem[b])

    def transpose(src, dst, nrows):
        for v in range(nrows):
            vvec = lanes * 0 + v
            for k in range(D // 16):
                vals = plsc.load_gather(src, [k * 16 + lanes, vvec])
                dst[v, pl.ds(k * 16, 16)] = vals

    @pl.when(c0 < NB)
    def _():
        in_copy(c0, 0).start()

    @pl.when(c0 + 1 < NB)
    def _():
        in_copy(c0 + 1, 1).start()

    def body(jj, carry):
        for b in range(2):
            j = 2 * jj + b
            c = c0 + j

            @pl.when(jnp.logical_and(j < BPW, c < NB))
            def _():
                in_copy(c, b).wait()

                @pl.when(j >= 2)
                def _():
                    out_copy(c - 2, b).wait()

                transpose(ibufs.at[b], obufs.at[b], W)

                @pl.when(jnp.logical_and(j + 2 < BPW, c + 2 < NB))
                def _():
                    in_copy(c + 2, b).start()

                out_copy(c, b).start()

        return carry

    lax.fori_loop(0, BPW // 2 + 1, body, 0)

    out_copy(c0, 0).wait()
    out_copy(c0, 1).wait()

    @pl.when(wid == NW - 1)
    def _():
        pltpu.sync_copy(wt_hbm.at[:, pl.ds(NB * W, TAIL)], tbuf)
        transpose(tbuf, obufs.at[0], TAIL)
        pltpu.sync_copy(obufs.at[0, pl.ds(0, TAIL)],
                        fmt_hbm.at[pl.ds(NB * W, TAIL)])


@functools.partial(
    pl.kernel,
    mesh=_mesh,
    compiler_params=_params,
    out_type=jax.ShapeDtypeStruct((N, 2 * D), jnp.float32),
    scratch_types=[
        pltpu.VMEM((CHUNKS_W, CHUNK), jnp.int32),
        pltpu.VMEM((3, CHUNK, 2 * D), jnp.float32),
        pltpu.SemaphoreType.DMA,
        pltpu.SemaphoreType.DMA,
        pltpu.SemaphoreType.DMA,
        pltpu.SemaphoreType.DMA,
        pltpu.SemaphoreType.DMA,
        pltpu.SemaphoreType.DMA,
    ],
)
def _gather(idx_hbm, table_hbm, out_hbm, idx_v, bufs,
            g0, g1, g2, w0, w1, w2):
    wid = lax.axis_index("s") * NC + lax.axis_index("c")
    chunk0 = wid * CHUNKS_W
    gsem = (g0, g1, g2)
    wsem = (w0, w1, w2)
    pltpu.sync_copy(idx_hbm.at[wid], idx_v)

    def start_gather(j, b):
        pltpu.async_copy(table_hbm.at[idx_v.at[j]], bufs.at[b], gsem[b])

    def drain_gather(j, b):
        pltpu.make_async_copy(
            table_hbm.at[idx_v.at[j]], bufs.at[b], gsem[b]).wait()

    def writeback(j, b):
        return pltpu.make_async_copy(
            bufs.at[b],
            out_hbm.at[pl.ds((chunk0 + j) * CHUNK, CHUNK)],
            wsem[b])

    start_gather(0, 0)
    start_gather(1, 1)

    def body(jj, carry):
        for t in range(3):
            j = 3 * jj + t

            @pl.when(j < CHUNKS_W)
            def _():
                drain_gather(j, t)
                writeback(j, t).start()

                @pl.when(j >= 1)
                def _():
                    writeback(j - 1, (t + 2) % 3).wait()

                @pl.when(j + 2 < CHUNKS_W)
                def _():
                    start_gather(j + 2, (t + 2) % 3)

        return carry

    lax.fori_loop(0, CHUNKS_W // 3 + 1, body, 0)
    writeback(CHUNKS_W - 1, (CHUNKS_W - 1) % 3).wait()


def _mask_body(x_ref, o_ref):
    o_ref[...] = (x_ref[...] != 0).astype(jnp.float32)


_mask_call = pl.pallas_call(
    _mask_body,
    out_shape=jax.ShapeDtypeStruct((B, S), jnp.float32),
)


def kernel(x, weight):
    wt = jnp.transpose(weight)
    fmt = _format(wt)
    xt = jnp.transpose(x).reshape(NW, CHUNKS_W, CHUNK).astype(jnp.int32)
    out = _gather(xt, fmt)
    mask = _mask_call(x)
    return out[:, :D].reshape(S, B, D), mask


# speedup vs baseline: 3.8782x; 3.8782x over previous
import functools

import jax
import jax.numpy as jnp
from jax import lax
from jax.experimental import pallas as pl
from jax.experimental.pallas import tpu as pltpu
from jax.experimental.pallas import tpu_sc as plsc

B = 1024
S = 200
D = 64
N = B * S
V = 1000000
NC = 2
NS = 16
NW = NC * NS
W = 128
NB = V // W
TAIL = V - NB * W
BPW = (NB + NW - 1) // NW
CHUNK = 128
PER_W = N // NW
CHUNKS_W = PER_W // CHUNK

_mesh = plsc.VectorSubcoreMesh(core_axis_name="c", subcore_axis_name="s")
_params = pltpu.CompilerParams(
    use_tc_tiling_on_sc=True, needs_layout_passes=False)


@functools.partial(
    pl.kernel,
    mesh=_mesh,
    compiler_params=_params,
    out_type=jax.ShapeDtypeStruct((V, 2 * D), jnp.float32),
    scratch_types=[
        pltpu.VMEM((2, D, W), jnp.float32),
        pltpu.VMEM((2, W, 2 * D), jnp.float32),
        pltpu.VMEM((D, TAIL), jnp.float32),
        pltpu.SemaphoreType.DMA,
        pltpu.SemaphoreType.DMA,
        pltpu.SemaphoreType.DMA,
        pltpu.SemaphoreType.DMA,
    ],
)
def _format(wt_hbm, fmt_hbm, ibufs, obufs, tbuf, gi0, gi1, go0, go1):
    wid = lax.axis_index("s") * NC + lax.axis_index("c")
    c0 = wid * BPW
    gisem = (gi0, gi1)
    gosem = (go0, go1)
    lanes = lax.iota(jnp.int32, 16)

    def in_copy(c, b):
        return pltpu.make_async_copy(
            wt_hbm.at[:, pl.ds(c * W, W)], ibufs.at[b], gisem[b])

    def out_copy(c, b):
        return pltpu.make_async_copy(
            obufs.at[b], fmt_hbm.at[pl.ds(c * W, W)], gosem[b])

    def transpose(src, dst, nrows):
        @functools.partial(plsc.parallel_loop, 0, nrows, unroll=8)
        def _(v):
            vvec = lanes * 0 + v
            for k in range(D // 16):
                vals = plsc.load_gather(src, [k * 16 + lanes, vvec])
                dst[v, pl.ds(k * 16, 16)] = vals

    @pl.when(c0 < NB)
    def _():
        in_copy(c0, 0).start()

    @pl.when(c0 + 1 < NB)
    def _():
        in_copy(c0 + 1, 1).start()

    def body(jj, carry):
        for b in range(2):
            j = 2 * jj + b
            c = c0 + j

            @pl.when(jnp.logical_and(j < BPW, c < NB))
            def _():
                in_copy(c, b).wait()

                @pl.when(j >= 2)
                def _():
                    out_copy(c - 2, b).wait()

                transpose(ibufs.at[b], obufs.at[b], W)

                @pl.when(jnp.logical_and(j + 2 < BPW, c + 2 < NB))
                def _():
                    in_copy(c + 2, b).start()

                out_copy(c, b).start()

        return carry

    lax.fori_loop(0, BPW // 2 + 1, body, 0)

    out_copy(c0, 0).wait()
    out_copy(c0, 1).wait()

    @pl.when(wid == NW - 1)
    def _():
        pltpu.sync_copy(wt_hbm.at[:, pl.ds(NB * W, TAIL)], tbuf)
        transpose(tbuf, obufs.at[0], TAIL)
        pltpu.sync_copy(obufs.at[0, pl.ds(0, TAIL)],
                        fmt_hbm.at[pl.ds(NB * W, TAIL)])


@functools.partial(
    pl.kernel,
    mesh=_mesh,
    compiler_params=_params,
    out_type=jax.ShapeDtypeStruct((N, 2 * D), jnp.float32),
    scratch_types=[
        pltpu.VMEM((CHUNKS_W, CHUNK), jnp.int32),
        pltpu.VMEM((3, CHUNK, 2 * D), jnp.float32),
        pltpu.SemaphoreType.DMA,
        pltpu.SemaphoreType.DMA,
        pltpu.SemaphoreType.DMA,
        pltpu.SemaphoreType.DMA,
        pltpu.SemaphoreType.DMA,
        pltpu.SemaphoreType.DMA,
    ],
)
def _gather(idx_hbm, table_hbm, out_hbm, idx_v, bufs,
            g0, g1, g2, w0, w1, w2):
    wid = lax.axis_index("s") * NC + lax.axis_index("c")
    chunk0 = wid * CHUNKS_W
    gsem = (g0, g1, g2)
    wsem = (w0, w1, w2)
    pltpu.sync_copy(idx_hbm.at[wid], idx_v)

    def start_gather(j, b):
        pltpu.async_copy(table_hbm.at[idx_v.at[j]], bufs.at[b], gsem[b])

    def drain_gather(j, b):
        pltpu.make_async_copy(
            table_hbm.at[idx_v.at[j]], bufs.at[b], gsem[b]).wait()

    def writeback(j, b):
        return pltpu.make_async_copy(
            bufs.at[b],
            out_hbm.at[pl.ds((chunk0 + j) * CHUNK, CHUNK)],
            wsem[b])

    start_gather(0, 0)
    start_gather(1, 1)

    def body(jj, carry):
        for t in range(3):
            j = 3 * jj + t

            @pl.when(j < CHUNKS_W)
            def _():
                drain_gather(j, t)
                writeback(j, t).start()

                @pl.when(j >= 1)
                def _():
                    writeback(j - 1, (t + 2) % 3).wait()

                @pl.when(j + 2 < CHUNKS_W)
                def _():
                    start_gather(j + 2, (t + 2) % 3)

        return carry

    lax.fori_loop(0, CHUNKS_W // 3 + 1, body, 0)
    writeback(CHUNKS_W - 1, (CHUNKS_W - 1) % 3).wait()


def _mask_body(x_ref, o_ref):
    o_ref[...] = (x_ref[...] != 0).astype(jnp.float32)


_mask_call = pl.pallas_call(
    _mask_body,
    out_shape=jax.ShapeDtypeStruct((B, S), jnp.float32),
)


def kernel(x, weight):
    wt = jnp.transpose(weight)
    fmt = _format(wt)
    xt = jnp.transpose(x).reshape(NW, CHUNKS_W, CHUNK).astype(jnp.int32)
    out = _gather(xt, fmt)
    mask = _mask_call(x)
    return out[:, :D].reshape(S, B, D), mask


# speedup vs baseline: 6.0866x; 1.5694x over previous
import functools

import jax
import jax.numpy as jnp
from jax import lax
from jax.experimental import pallas as pl
from jax.experimental.pallas import tpu as pltpu
from jax.experimental.pallas import tpu_sc as plsc

B = 1024
S = 200
D = 64
N = B * S
V = 1000000
NC = 2
NS = 16
NW = NC * NS
W = 128
NB = V // W
TAIL = V - NB * W
BPW = (NB + NW - 1) // NW
CHUNK = 128
PER_W = N // NW
CHUNKS_W = PER_W // CHUNK
CPS = B // CHUNK

_mesh = plsc.VectorSubcoreMesh(core_axis_name="c", subcore_axis_name="s")
_params = pltpu.CompilerParams(
    use_tc_tiling_on_sc=True, needs_layout_passes=False)


@functools.partial(
    pl.kernel,
    mesh=_mesh,
    compiler_params=_params,
    out_type=jax.ShapeDtypeStruct((V // 2, 2 * D), jnp.float32),
    scratch_types=[
        pltpu.VMEM((2, D, W), jnp.float32),
        pltpu.VMEM((2, W // 2, 2 * D), jnp.float32),
        pltpu.VMEM((D, TAIL), jnp.float32),
        pltpu.SemaphoreType.DMA,
        pltpu.SemaphoreType.DMA,
        pltpu.SemaphoreType.DMA,
        pltpu.SemaphoreType.DMA,
    ],
)
def _format(wt_hbm, fmt_hbm, ibufs, obufs, tbuf, gi0, gi1, go0, go1):
    wid = lax.axis_index("s") * NC + lax.axis_index("c")
    c0 = wid * BPW
    gisem = (gi0, gi1)
    gosem = (go0, go1)
    lanes = lax.iota(jnp.int32, 16)

    def in_copy(c, b):
        return pltpu.make_async_copy(
            wt_hbm.at[:, pl.ds(c * W, W)], ibufs.at[b], gisem[b])

    def out_copy(c, b):
        return pltpu.make_async_copy(
            obufs.at[b], fmt_hbm.at[pl.ds(c * (W // 2), W // 2)], gosem[b])

    def transpose(src, dst, nrows):
        @functools.partial(plsc.parallel_loop, 0, nrows // 2, unroll=8)
        def _(p):
            for h in range(2):
                vvec = lanes * 0 + (2 * p + h)
                for k in range(D // 16):
                    vals = plsc.load_gather(src, [k * 16 + lanes, vvec])
                    dst[p, pl.ds(h * D + k * 16, 16)] = vals

    @pl.when(c0 < NB)
    def _():
        in_copy(c0, 0).start()

    @pl.when(c0 + 1 < NB)
    def _():
        in_copy(c0 + 1, 1).start()

    def body(jj, carry):
        for b in range(2):
            j = 2 * jj + b
            c = c0 + j

            @pl.when(jnp.logical_and(j < BPW, c < NB))
            def _():
                in_copy(c, b).wait()

                @pl.when(j >= 2)
                def _():
                    out_copy(c - 2, b).wait()

                transpose(ibufs.at[b], obufs.at[b], W)

                @pl.when(jnp.logical_and(j + 2 < BPW, c + 2 < NB))
                def _():
                    in_copy(c + 2, b).start()

                out_copy(c, b).start()

        return carry

    lax.fori_loop(0, BPW // 2 + 1, body, 0)

    out_copy(c0, 0).wait()
    out_copy(c0, 1).wait()

    @pl.when(wid == NW - 1)
    def _():
        pltpu.sync_copy(wt_hbm.at[:, pl.ds(NB * W, TAIL)], tbuf)
        transpose(tbuf, obufs.at[0], TAIL)
        pltpu.sync_copy(obufs.at[0, pl.ds(0, TAIL // 2)],
                        fmt_hbm.at[pl.ds(NB * W // 2, TAIL // 2)])


@functools.partial(
    pl.kernel,
    mesh=_mesh,
    compiler_params=_params,
    out_type=jax.ShapeDtypeStruct((S, D, B), jnp.float32),
    scratch_types=[
        pltpu.VMEM((CHUNKS_W, CHUNK), jnp.int32),
        pltpu.VMEM((CHUNKS_W, CHUNK), jnp.int32),
        pltpu.VMEM((2, CHUNK, 2 * D), jnp.float32),
        pltpu.VMEM((2, D, CHUNK), jnp.float32),
        pltpu.SemaphoreType.DMA,
        pltpu.SemaphoreType.DMA,
        pltpu.SemaphoreType.DMA,
        pltpu.SemaphoreType.DMA,
    ],
)
def _gather(idx_hbm, table_hbm, out_hbm, idx_v, idx2_v, bufs, packs,
            g0, g1, w0, w1):
    wid = lax.axis_index("s") * NC + lax.axis_index("c")
    chunk0 = wid * CHUNKS_W
    gsem = (g0, g1)
    wsem = (w0, w1)
    lanes = lax.iota(jnp.int32, 16)
    pltpu.sync_copy(idx_hbm.at[wid], idx_v)

    def shift_body(j, carry):
        for g in range(CHUNK // 16):
            v = idx_v[j, pl.ds(g * 16, 16)]
            idx2_v[j, pl.ds(g * 16, 16)] = lax.shift_right_logical(v, 1)
        return carry

    lax.fori_loop(0, CHUNKS_W, shift_body, 0)

    def start_gather(j, b):
        pltpu.async_copy(table_hbm.at[idx2_v.at[j]], bufs.at[b], gsem[b])

    def drain_gather(j, b):
        pltpu.make_async_copy(
            table_hbm.at[idx2_v.at[j]], bufs.at[b], gsem[b]).wait()

    def compact(j, b):
        buf = bufs.at[b]

        @functools.partial(plsc.parallel_loop, 0, CHUNK // 16, unroll=4)
        def _(g):
            rows = g * 16 + lanes
            half = lax.mul(
                lax.bitwise_and(idx_v[j, pl.ds(g * 16, 16)], 1), D)
            for d in range(D):
                vals = plsc.load_gather(buf, [rows, half + d])
                packs[b, d, pl.ds(g * 16, 16)] = vals

    def writeback(j, b):
        c_g = chunk0 + j
        s_i = lax.div(c_g, CPS)
        b0 = lax.mul(lax.rem(c_g, CPS), CHUNK)
        return pltpu.make_async_copy(
            packs.at[b], out_hbm.at[s_i, :, pl.ds(b0, CHUNK)], wsem[b])

    start_gather(0, 0)
    start_gather(1, 1)

    def body(jj, carry):
        for b in range(2):
            j = 2 * jj + b
            drain_gather(j, b)

            @pl.when(j >= 2)
            def _():
                writeback(j - 2, b).wait()

            compact(j, b)

            @pl.when(j + 2 < CHUNKS_W)
            def _():
                start_gather(j + 2, b)

            writeback(j, b).start()
        return carry

    lax.fori_loop(0, CHUNKS_W // 2, body, 0)
    writeback(CHUNKS_W - 2, 0).wait()
    writeback(CHUNKS_W - 1, 1).wait()


def _mask_body(x_ref, o_ref):
    o_ref[...] = (x_ref[...] != 0).astype(jnp.float32)


_mask_call = pl.pallas_call(
    _mask_body,
    out_shape=jax.ShapeDtypeStruct((B, S), jnp.float32),
)


def kernel(x, weight):
    wt = jnp.transpose(weight)
    fmt = _format(wt)
    xt = jnp.transpose(x).reshape(NW, CHUNKS_W, CHUNK).astype(jnp.int32)
    out = _gather(xt, fmt)
    mask = _mask_call(x)
    return jnp.transpose(out, (0, 2, 1)), mask


# speedup vs baseline: 6.0993x; 1.0021x over previous
import functools

import jax
import jax.numpy as jnp
from jax import lax
from jax.experimental import pallas as pl
from jax.experimental.pallas import tpu as pltpu
from jax.experimental.pallas import tpu_sc as plsc

B = 1024
S = 200
D = 64
N = B * S
V = 1000000
NC = 2
NS = 16
NW = NC * NS
W = 128
NB = V // W
TAIL = V - NB * W
BPW = (NB + NW - 1) // NW
CHUNK = 128
PER_W = N // NW
CHUNKS_W = PER_W // CHUNK
CPS = B // CHUNK

_mesh = plsc.VectorSubcoreMesh(core_axis_name="c", subcore_axis_name="s")
_params = pltpu.CompilerParams(
    use_tc_tiling_on_sc=True, needs_layout_passes=False)


@functools.partial(
    pl.kernel,
    mesh=_mesh,
    compiler_params=_params,
    out_type=jax.ShapeDtypeStruct((V // 2, 2 * D), jnp.float32),
    scratch_types=[
        pltpu.VMEM((2, D, W), jnp.float32),
        pltpu.VMEM((2, W // 2, 2 * D), jnp.float32),
        pltpu.VMEM((D, TAIL), jnp.float32),
        pltpu.SemaphoreType.DMA,
        pltpu.SemaphoreType.DMA,
        pltpu.SemaphoreType.DMA,
        pltpu.SemaphoreType.DMA,
    ],
)
def _format(wt_hbm, fmt_hbm, ibufs, obufs, tbuf, gi0, gi1, go0, go1):
    wid = lax.axis_index("s") * NC + lax.axis_index("c")
    c0 = wid * BPW
    gisem = (gi0, gi1)
    gosem = (go0, go1)
    lanes = lax.iota(jnp.int32, 16)

    def in_copy(c, b):
        return pltpu.make_async_copy(
            wt_hbm.at[:, pl.ds(c * W, W)], ibufs.at[b], gisem[b])

    def out_copy(c, b):
        return pltpu.make_async_copy(
            obufs.at[b], fmt_hbm.at[pl.ds(c * (W // 2), W // 2)], gosem[b])

    def transpose(src, dst, nrows):
        @functools.partial(plsc.parallel_loop, 0, nrows // 2, unroll=8)
        def _(p):
            for h in range(2):
                vvec = lanes * 0 + (2 * p + h)
                for k in range(D // 16):
                    vals = plsc.load_gather(src, [k * 16 + lanes, vvec])
                    dst[p, pl.ds(h * D + k * 16, 16)] = vals

    @pl.when(c0 < NB)
    def _():
        in_copy(c0, 0).start()

    @pl.when(c0 + 1 < NB)
    def _():
        in_copy(c0 + 1, 1).start()

    def body(jj, carry):
        for b in range(2):
            j = 2 * jj + b
            c = c0 + j

            @pl.when(jnp.logical_and(j < BPW, c < NB))
            def _():
                in_copy(c, b).wait()

                @pl.when(j >= 2)
                def _():
                    out_copy(c - 2, b).wait()

                transpose(ibufs.at[b], obufs.at[b], W)

                @pl.when(jnp.logical_and(j + 2 < BPW, c + 2 < NB))
                def _():
                    in_copy(c + 2, b).start()

                out_copy(c, b).start()

        return carry

    lax.fori_loop(0, BPW // 2 + 1, body, 0)

    out_copy(c0, 0).wait()
    out_copy(c0, 1).wait()

    @pl.when(wid == NW - 1)
    def _():
        pltpu.sync_copy(wt_hbm.at[:, pl.ds(NB * W, TAIL)], tbuf)
        transpose(tbuf, obufs.at[0], TAIL)
        pltpu.sync_copy(obufs.at[0, pl.ds(0, TAIL // 2)],
                        fmt_hbm.at[pl.ds(NB * W // 2, TAIL // 2)])


@functools.partial(
    pl.kernel,
    mesh=_mesh,
    compiler_params=_params,
    out_type=jax.ShapeDtypeStruct((S, D, B), jnp.float32),
    scratch_types=[
        pltpu.VMEM((CHUNKS_W, CHUNK), jnp.int32),
        pltpu.VMEM((CHUNKS_W, CHUNK), jnp.int32),
        pltpu.VMEM((3, CHUNK, 2 * D), jnp.float32),
        pltpu.VMEM((3, D, CHUNK), jnp.float32),
        pltpu.SemaphoreType.DMA,
        pltpu.SemaphoreType.DMA,
        pltpu.SemaphoreType.DMA,
        pltpu.SemaphoreType.DMA,
        pltpu.SemaphoreType.DMA,
        pltpu.SemaphoreType.DMA,
    ],
)
def _gather(idx_hbm, table_hbm, out_hbm, idx_v, idx2_v, bufs, packs,
            g0, g1, g2, w0, w1, w2):
    wid = lax.axis_index("s") * NC + lax.axis_index("c")
    chunk0 = wid * CHUNKS_W
    gsem = (g0, g1, g2)
    wsem = (w0, w1, w2)
    lanes = lax.iota(jnp.int32, 16)
    pltpu.sync_copy(idx_hbm.at[wid], idx_v)

    def shift_body(j, carry):
        for g in range(CHUNK // 16):
            v = idx_v[j, pl.ds(g * 16, 16)]
            idx2_v[j, pl.ds(g * 16, 16)] = lax.shift_right_logical(v, 1)
        return carry

    lax.fori_loop(0, CHUNKS_W, shift_body, 0)

    def start_gather(j, b):
        pltpu.async_copy(table_hbm.at[idx2_v.at[j]], bufs.at[b], gsem[b])

    def drain_gather(j, b):
        pltpu.make_async_copy(
            table_hbm.at[idx2_v.at[j]], bufs.at[b], gsem[b]).wait()

    def compact(j, b):
        buf = bufs.at[b]
        rows_l = [g * 16 + lanes for g in range(CHUNK // 16)]
        half_l = [
            lax.mul(lax.bitwise_and(idx_v[j, pl.ds(g * 16, 16)], 1), D)
            for g in range(CHUNK // 16)
        ]

        @functools.partial(plsc.parallel_loop, 0, D, unroll=8)
        def _(d):
            for g in range(CHUNK // 16):
                vals = plsc.load_gather(buf, [rows_l[g], half_l[g] + d])
                packs[b, d, pl.ds(g * 16, 16)] = vals

    def writeback(j, b):
        c_g = chunk0 + j
        s_i = lax.div(c_g, CPS)
        b0 = lax.mul(lax.rem(c_g, CPS), CHUNK)
        return pltpu.make_async_copy(
            packs.at[b], out_hbm.at[s_i, :, pl.ds(b0, CHUNK)], wsem[b])

    start_gather(0, 0)
    start_gather(1, 1)

    def body(jj, carry):
        for t in range(3):
            j = 3 * jj + t

            @pl.when(j < CHUNKS_W)
            def _():
                drain_gather(j, t)

                @pl.when(j >= 3)
                def _():
                    writeback(j - 3, t).wait()

                compact(j, t)

                @pl.when(j + 2 < CHUNKS_W)
                def _():
                    start_gather(j + 2, (t + 2) % 3)

                writeback(j, t).start()
        return carry

    lax.fori_loop(0, CHUNKS_W // 3 + 1, body, 0)
    writeback(CHUNKS_W - 3, (CHUNKS_W - 3) % 3).wait()
    writeback(CHUNKS_W - 2, (CHUNKS_W - 2) % 3).wait()
    writeback(CHUNKS_W - 1, (CHUNKS_W - 1) % 3).wait()


def _mask_body(x_ref, o_ref):
    o_ref[...] = (x_ref[...] != 0).astype(jnp.float32)


_mask_call = pl.pallas_call(
    _mask_body,
    out_shape=jax.ShapeDtypeStruct((B, S), jnp.float32),
)


def kernel(x, weight):
    wt = jnp.transpose(weight)
    fmt = _format(wt)
    xt = jnp.transpose(x).reshape(NW, CHUNKS_W, CHUNK).astype(jnp.int32)
    out = _gather(xt, fmt)
    mask = _mask_call(x)
    return jnp.transpose(out, (0, 2, 1)), mask


# speedup vs baseline: 6.4493x; 1.0574x over previous
import functools

import jax
import jax.numpy as jnp
from jax import lax
from jax.experimental import pallas as pl
from jax.experimental.pallas import tpu as pltpu
from jax.experimental.pallas import tpu_sc as plsc

B = 1024
S = 200
D = 64
N = B * S
V = 1000000
NC = 2
NS = 16
NW = NC * NS
W = 256
NB = V // W
TAIL = V - NB * W
BPW = (NB + NW - 1) // NW
CHUNK = 128
PER_W = N // NW
CHUNKS_W = PER_W // CHUNK
CPS = B // CHUNK

_mesh = plsc.VectorSubcoreMesh(core_axis_name="c", subcore_axis_name="s")
_params = pltpu.CompilerParams(
    use_tc_tiling_on_sc=True, needs_layout_passes=False)


@functools.partial(
    pl.kernel,
    mesh=_mesh,
    compiler_params=_params,
    out_type=jax.ShapeDtypeStruct((V // 2, 2 * D), jnp.float32),
    scratch_types=[
        pltpu.VMEM((2, D, W), jnp.float32),
        pltpu.VMEM((2, W // 2, 2 * D), jnp.float32),
        pltpu.VMEM((D, TAIL), jnp.float32),
        pltpu.SemaphoreType.DMA,
        pltpu.SemaphoreType.DMA,
        pltpu.SemaphoreType.DMA,
        pltpu.SemaphoreType.DMA,
    ],
)
def _format(wt_hbm, fmt_hbm, ibufs, obufs, tbuf, gi0, gi1, go0, go1):
    wid = lax.axis_index("s") * NC + lax.axis_index("c")
    c0 = wid * BPW
    gisem = (gi0, gi1)
    gosem = (go0, go1)
    lanes = lax.iota(jnp.int32, 16)

    def in_copy(c, b):
        return pltpu.make_async_copy(
            wt_hbm.at[:, pl.ds(c * W, W)], ibufs.at[b], gisem[b])

    def out_copy(c, b):
        return pltpu.make_async_copy(
            obufs.at[b], fmt_hbm.at[pl.ds(c * (W // 2), W // 2)], gosem[b])

    def transpose(src, dst, nrows):
        @functools.partial(plsc.parallel_loop, 0, nrows // 2, unroll=8)
        def _(p):
            for h in range(2):
                vvec = lanes * 0 + (2 * p + h)
                for k in range(D // 16):
                    vals = plsc.load_gather(src, [k * 16 + lanes, vvec])
                    dst[p, pl.ds(h * D + k * 16, 16)] = vals

    @pl.when(c0 < NB)
    def _():
        in_copy(c0, 0).start()

    @pl.when(c0 + 1 < NB)
    def _():
        in_copy(c0 + 1, 1).start()

    def body(jj, carry):
        for b in range(2):
            j = 2 * jj + b
            c = c0 + j

            @pl.when(jnp.logical_and(j < BPW, c < NB))
            def _():
                in_copy(c, b).wait()

                @pl.when(j >= 2)
                def _():
                    out_copy(c - 2, b).wait()

                transpose(ibufs.at[b], obufs.at[b], W)

                @pl.when(jnp.logical_and(j + 2 < BPW, c + 2 < NB))
                def _():
                    in_copy(c + 2, b).start()

                out_copy(c, b).start()

        return carry

    lax.fori_loop(0, BPW // 2 + 1, body, 0)

    out_copy(c0, 0).wait()
    out_copy(c0, 1).wait()

    @pl.when(wid == NW - 1)
    def _():
        pltpu.sync_copy(wt_hbm.at[:, pl.ds(NB * W, TAIL)], tbuf)
        transpose(tbuf, obufs.at[0], TAIL)
        pltpu.sync_copy(obufs.at[0, pl.ds(0, TAIL // 2)],
                        fmt_hbm.at[pl.ds(NB * W // 2, TAIL // 2)])


@functools.partial(
    pl.kernel,
    mesh=_mesh,
    compiler_params=_params,
    out_type=jax.ShapeDtypeStruct((S, D, B), jnp.float32),
    scratch_types=[
        pltpu.VMEM((CHUNKS_W, CHUNK), jnp.int32),
        pltpu.VMEM((CHUNKS_W, CHUNK), jnp.int32),
        pltpu.VMEM((3, CHUNK, 2 * D), jnp.float32),
        pltpu.VMEM((3, D, CHUNK), jnp.float32),
        pltpu.SemaphoreType.DMA,
        pltpu.SemaphoreType.DMA,
        pltpu.SemaphoreType.DMA,
        pltpu.SemaphoreType.DMA,
        pltpu.SemaphoreType.DMA,
        pltpu.SemaphoreType.DMA,
    ],
)
def _gather(idx_hbm, table_hbm, out_hbm, idx_v, idx2_v, bufs, packs,
            g0, g1, g2, w0, w1, w2):
    wid = lax.axis_index("s") * NC + lax.axis_index("c")
    chunk0 = wid * CHUNKS_W
    gsem = (g0, g1, g2)
    wsem = (w0, w1, w2)
    lanes = lax.iota(jnp.int32, 16)
    pltpu.sync_copy(idx_hbm.at[wid], idx_v)

    def shift_body(j, carry):
        for g in range(CHUNK // 16):
            v = idx_v[j, pl.ds(g * 16, 16)]
            idx2_v[j, pl.ds(g * 16, 16)] = lax.shift_right_logical(v, 1)
        return carry

    lax.fori_loop(0, CHUNKS_W, shift_body, 0)

    def start_gather(j, b):
        pltpu.async_copy(table_hbm.at[idx2_v.at[j]], bufs.at[b], gsem[b])

    def drain_gather(j, b):
        pltpu.make_async_copy(
            table_hbm.at[idx2_v.at[j]], bufs.at[b], gsem[b]).wait()

    def compact(j, b):
        buf = bufs.at[b]
        rows_l = [g * 16 + lanes for g in range(CHUNK // 16)]
        half_l = [
            lax.mul(lax.bitwise_and(idx_v[j, pl.ds(g * 16, 16)], 1), D)
            for g in range(CHUNK // 16)
        ]

        @functools.partial(plsc.parallel_loop, 0, D, unroll=8)
        def _(d):
            for g in range(CHUNK // 16):
                vals = plsc.load_gather(buf, [rows_l[g], half_l[g] + d])
                packs[b, d, pl.ds(g * 16, 16)] = vals

    def writeback(j, b):
        c_g = chunk0 + j
        s_i = lax.div(c_g, CPS)
        b0 = lax.mul(lax.rem(c_g, CPS), CHUNK)
        return pltpu.make_async_copy(
            packs.at[b], out_hbm.at[s_i, :, pl.ds(b0, CHUNK)], wsem[b])

    start_gather(0, 0)
    start_gather(1, 1)

    def body(jj, carry):
        for t in range(3):
            j = 3 * jj + t

            @pl.when(j < CHUNKS_W)
            def _():
                drain_gather(j, t)

                @pl.when(j >= 3)
                def _():
                    writeback(j - 3, t).wait()

                compact(j, t)

                @pl.when(j + 2 < CHUNKS_W)
                def _():
                    start_gather(j + 2, (t + 2) % 3)

                writeback(j, t).start()
        return carry

    lax.fori_loop(0, CHUNKS_W // 3 + 1, body, 0)
    writeback(CHUNKS_W - 3, (CHUNKS_W - 3) % 3).wait()
    writeback(CHUNKS_W - 2, (CHUNKS_W - 2) % 3).wait()
    writeback(CHUNKS_W - 1, (CHUNKS_W - 1) % 3).wait()


def _mask_body(x_ref, o_ref):
    o_ref[...] = (x_ref[...] != 0).astype(jnp.float32)


_mask_call = pl.pallas_call(
    _mask_body,
    out_shape=jax.ShapeDtypeStruct((B, S), jnp.float32),
)


def kernel(x, weight):
    wt = jnp.transpose(weight)
    fmt = _format(wt)
    xt = jnp.transpose(x).reshape(NW, CHUNKS_W, CHUNK).astype(jnp.int32)
    out = _gather(xt, fmt)
    mask = _mask_call(x)
    return jnp.transpose(out, (0, 2, 1)), mask
